# Initial kernel scaffold; baseline (speedup 1.0000x reference)
import jax
import jax.numpy as jnp
from jax.experimental import pallas as pl


def kernel(mat, proj_dir):
    raise NotImplementedError("write your pallas kernel here")


# speedup vs baseline: 11.0986x; 11.0986x over previous
import jax
import jax.numpy as jnp
from jax.experimental import pallas as pl

_NUM_PROJS = 16
_CHUNK = 1024


def _lsh_body(mat_ref, proj_ref, out_ref):
    x = mat_ref[0]
    p = proj_ref[0]
    y = jax.lax.dot_general(
        p, x, (((0,), (1,)), ((), ())),
        preferred_element_type=jnp.float32,
    )
    bits = (y > 0).astype(jnp.int32)
    enc = jnp.left_shift(
        jnp.int32(1),
        jax.lax.broadcasted_iota(jnp.int32, (_NUM_PROJS, 1), 0),
    )
    b = jnp.sum(bits * enc, axis=0, keepdims=True)
    out_ref[0] = b ^ (b >> 1)


def kernel(mat, proj_dir):
    B, H, S, D = mat.shape
    n_row = S // _CHUNK
    grid = B * H * n_row
    mat_r = mat.reshape(grid, _CHUNK, D)
    proj_r = proj_dir.reshape(H, D, _NUM_PROJS)
    out = pl.pallas_call(
        _lsh_body,
        grid=(grid,),
        in_specs=[
            pl.BlockSpec((1, _CHUNK, D), lambda i: (i, 0, 0)),
            pl.BlockSpec((1, D, _NUM_PROJS), lambda i: ((i // n_row) % H, 0, 0)),
        ],
        out_specs=pl.BlockSpec((1, 1, _CHUNK), lambda i: (i, 0, 0)),
        out_shape=jax.ShapeDtypeStruct((grid, 1, _CHUNK), jnp.int32),
    )(mat_r, proj_r)
    return out.reshape(B, H, S)


# speedup vs baseline: 25.6767x; 2.3135x over previous
import jax
import jax.numpy as jnp
from jax.experimental import pallas as pl

_NUM_PROJS = 16
_CHUNK = 4096


def _lsh_body(mat_ref, proj_ref, out_ref):
    x = mat_ref[0]
    p = proj_ref[0]
    y = jax.lax.dot_general(
        p, x, (((0,), (1,)), ((), ())),
        preferred_element_type=jnp.float32,
    )
    bits = (y > 0).astype(jnp.int32)
    enc = jnp.left_shift(
        jnp.int32(1),
        jax.lax.broadcasted_iota(jnp.int32, (_NUM_PROJS, 1), 0),
    )
    b = jnp.sum(bits * enc, axis=0, keepdims=True)
    out_ref[0] = b ^ (b >> 1)


def kernel(mat, proj_dir):
    B, H, S, D = mat.shape
    n_row = S // _CHUNK
    grid = B * H * n_row
    mat_r = mat.reshape(grid, _CHUNK, D)
    proj_r = proj_dir.reshape(H, D, _NUM_PROJS)
    out = pl.pallas_call(
        _lsh_body,
        grid=(grid,),
        in_specs=[
            pl.BlockSpec((1, _CHUNK, D), lambda i: (i, 0, 0)),
            pl.BlockSpec((1, D, _NUM_PROJS), lambda i: ((i // n_row) % H, 0, 0)),
        ],
        out_specs=pl.BlockSpec((1, 1, _CHUNK), lambda i: (i, 0, 0)),
        out_shape=jax.ShapeDtypeStruct((grid, 1, _CHUNK), jnp.int32),
    )(mat_r, proj_r)
    return out.reshape(B, H, S)


# speedup vs baseline: 33.3001x; 1.2969x over previous
import jax
import jax.numpy as jnp
from jax.experimental import pallas as pl

_NUM_PROJS = 16
_HPB = 2


def _lsh_body(mat_ref, proj_ref, out_ref):
    x = mat_ref[0]
    p = proj_ref[0]
    y = jax.lax.dot_general(
        p, x, (((1,), (2,)), ((0,), (0,))),
        preferred_element_type=jnp.float32,
    )
    bits = (y > 0).astype(jnp.int32)
    enc = jnp.left_shift(
        jnp.int32(1),
        jax.lax.broadcasted_iota(jnp.int32, (1, _NUM_PROJS, 1), 1),
    )
    b = jnp.sum(bits * enc, axis=1)
    out_ref[0] = b ^ (b >> 1)


def kernel(mat, proj_dir):
    B, H, S, D = mat.shape
    grid = (B * H) // _HPB
    n_proj_grp = H // _HPB
    mat_r = mat.reshape(grid, _HPB, S, D)
    proj_r = proj_dir.reshape(n_proj_grp, _HPB, D, _NUM_PROJS)
    out = pl.pallas_call(
        _lsh_body,
        grid=(grid,),
        in_specs=[
            pl.BlockSpec((1, _HPB, S, D), lambda i: (i, 0, 0, 0)),
            pl.BlockSpec((1, _HPB, D, _NUM_PROJS),
                         lambda i: (i % n_proj_grp, 0, 0, 0)),
        ],
        out_specs=pl.BlockSpec((1, _HPB, S), lambda i: (i, 0, 0)),
        out_shape=jax.ShapeDtypeStruct((grid, _HPB, S), jnp.int32),
    )(mat_r, proj_r)
    return out.reshape(B, H, S)


# speedup vs baseline: 37.7400x; 1.1333x over previous
import jax
import jax.numpy as jnp
from jax.experimental import pallas as pl

_NUM_PROJS = 16
_HPB = 4


def _lsh_body(mat_ref, proj_ref, out_ref):
    x = mat_ref[0]
    p = proj_ref[0]
    y = jax.lax.dot_general(
        p, x, (((1,), (2,)), ((0,), (0,))),
        preferred_element_type=jnp.float32,
    )
    bits = (y > 0).astype(jnp.int32)
    enc = jnp.left_shift(
        jnp.int32(1),
        jax.lax.broadcasted_iota(jnp.int32, (1, _NUM_PROJS, 1), 1),
    )
    b = jnp.sum(bits * enc, axis=1)
    out_ref[0] = b ^ (b >> 1)


def kernel(mat, proj_dir):
    B, H, S, D = mat.shape
    grid = (B * H) // _HPB
    n_proj_grp = H // _HPB
    mat_r = mat.reshape(grid, _HPB, S, D)
    proj_r = proj_dir.reshape(n_proj_grp, _HPB, D, _NUM_PROJS)
    out = pl.pallas_call(
        _lsh_body,
        grid=(grid,),
        in_specs=[
            pl.BlockSpec((1, _HPB, S, D), lambda i: (i, 0, 0, 0)),
            pl.BlockSpec((1, _HPB, D, _NUM_PROJS),
                         lambda i: (i % n_proj_grp, 0, 0, 0)),
        ],
        out_specs=pl.BlockSpec((1, _HPB, S), lambda i: (i, 0, 0)),
        out_shape=jax.ShapeDtypeStruct((grid, _HPB, S), jnp.int32),
    )(mat_r, proj_r)
    return out.reshape(B, H, S)
